# baseline (speedup 1.0000x reference)
import jax
import jax.numpy as jnp
from jax.experimental import pallas as pl

_S = 500
_BLK = 4000


def _posadd_kernel(x_ref, pos_ref, o_ref):
    o_ref[...] = x_ref[...] + pos_ref[...]


def kernel(x, pos_table):
    B, S, D = x.shape
    x2 = x.reshape(B * S, D)
    reps = _BLK // S
    pos_tiled = jnp.tile(pos_table[:S], (reps, 1))
    out2 = pl.pallas_call(
        _posadd_kernel,
        grid=((B * S) // _BLK,),
        in_specs=[
            pl.BlockSpec((_BLK, D), lambda i: (i, 0)),
            pl.BlockSpec((_BLK, D), lambda i: (0, 0)),
        ],
        out_specs=pl.BlockSpec((_BLK, D), lambda i: (i, 0)),
        out_shape=jax.ShapeDtypeStruct((B * S, D), x.dtype),
    )(x2, pos_tiled)
    return out2.reshape(B, S, D)


# speedup vs baseline: 1.7999x; 1.7999x over previous
import jax
import jax.numpy as jnp
from jax.experimental import pallas as pl

_BB = 32


def _posadd_kernel(x_ref, pos_ref, o_ref):
    pos = pos_ref[0:500, :]
    o_ref[...] = x_ref[...] + pos[None, :, :]


def kernel(x, pos_table):
    B, S, D = x.shape
    return pl.pallas_call(
        _posadd_kernel,
        grid=(B // _BB,),
        in_specs=[
            pl.BlockSpec((_BB, S, D), lambda i: (i, 0, 0)),
            pl.BlockSpec(pos_table.shape, lambda i: (0, 0)),
        ],
        out_specs=pl.BlockSpec((_BB, S, D), lambda i: (i, 0, 0)),
        out_shape=jax.ShapeDtypeStruct((B, S, D), x.dtype),
    )(x, pos_table)


# speedup vs baseline: 1.8012x; 1.0007x over previous
import jax
import jax.numpy as jnp
from jax.experimental import pallas as pl
from jax.experimental.pallas import tpu as pltpu

_BB = 32


def _posadd_kernel(x_ref, pos_ref, o_ref):
    pos = pos_ref[0:500, :]
    o_ref[...] = x_ref[...] + pos[None, :, :]


def kernel(x, pos_table):
    B, S, D = x.shape
    return pl.pallas_call(
        _posadd_kernel,
        grid=(B // _BB,),
        in_specs=[
            pl.BlockSpec((_BB, S, D), lambda i: (i, 0, 0)),
            pl.BlockSpec(pos_table.shape, lambda i: (0, 0)),
        ],
        out_specs=pl.BlockSpec((_BB, S, D), lambda i: (i, 0, 0)),
        out_shape=jax.ShapeDtypeStruct((B, S, D), x.dtype),
        compiler_params=pltpu.CompilerParams(
            dimension_semantics=("parallel",),
        ),
    )(x, pos_table)
